# baseline (speedup 1.0000x reference)
import functools

import jax
import jax.numpy as jnp
from jax import lax
from jax.experimental import pallas as pl
from jax.experimental.pallas import tpu as pltpu
from jax.experimental.pallas import tpu_sc as plsc

N = 10000
E = 320000
D = 128
HID = 128
LAT = 64

NC = 2
NS = 16
NW = NC * NS


EPW = E // NW
GCH = 80
GNCH = EPW // GCH


@functools.partial(
    pl.kernel,
    out_type=jax.ShapeDtypeStruct((E, D), jnp.float32),
    mesh=plsc.VectorSubcoreMesh(core_axis_name="c", subcore_axis_name="s", num_cores=NC, num_subcores=NS),
    compiler_params=pltpu.CompilerParams(needs_layout_passes=False),
    scratch_types=[
        pltpu.VMEM((GCH,), jnp.int32),
        pltpu.VMEM((GCH,), jnp.int32),
        pltpu.VMEM((GCH, D), jnp.float32),
        pltpu.VMEM((GCH, D), jnp.float32),
        pltpu.SemaphoreType.DMA,
        pltpu.SemaphoreType.DMA,
    ],
)
def _sc_gather_add(a_hbm, b_hbm, dst_hbm, src_hbm, out_hbm,
                   di_v, si_v, abuf, bbuf, sem_a, sem_b):
    wid = lax.axis_index("s") * NC + lax.axis_index("c")
    base = wid * EPW

    def chunk(i, _):
        off = base + i * GCH
        pltpu.sync_copy(dst_hbm.at[pl.ds(off, GCH)], di_v)
        pltpu.sync_copy(src_hbm.at[pl.ds(off, GCH)], si_v)
        cp_a = pltpu.async_copy(a_hbm.at[di_v], abuf, sem_a)
        cp_b = pltpu.async_copy(b_hbm.at[si_v], bbuf, sem_b)
        cp_a.wait()
        cp_b.wait()

        def row(r, _):
            for c in range(D // 16):
                sl = pl.ds(c * 16, 16)
                abuf[r, sl] = abuf[r, sl] + bbuf[r, sl]
            return 0

        lax.fori_loop(0, GCH, row, 0)
        pltpu.sync_copy(abuf, out_hbm.at[pl.ds(off, GCH)])
        return 0

    lax.fori_loop(0, GNCH, chunk, 0)


NPW = 320
NOUT = NW * NPW
SCH = 2560
SNCH = E // SCH
GRP = 128


@functools.partial(
    pl.kernel,
    out_type=jax.ShapeDtypeStruct((NOUT, LAT), jnp.float32),
    mesh=plsc.VectorSubcoreMesh(core_axis_name="c", subcore_axis_name="s", num_cores=NC, num_subcores=NS),
    compiler_params=pltpu.CompilerParams(needs_layout_passes=False, use_tc_tiling_on_sc=False),
    scratch_types=[
        pltpu.VMEM((NPW + 1, LAT), jnp.float32),
        pltpu.VMEM((SCH,), jnp.int32),
        pltpu.VMEM((SCH + 16,), jnp.int32),
        pltpu.VMEM((SCH + 16,), jnp.int32),
        pltpu.VMEM((GRP, LAT), jnp.float32),
        pltpu.SemaphoreType.DMA,
    ],
)
def _sc_scatter_max(dst_hbm, m_hbm, out_hbm, acc, dch, eids, dvals, mbuf, sem):
    wid = lax.axis_index("s") * NC + lax.axis_index("c")
    lo = wid * NPW

    def zero_acc(r, _):
        for c in range(LAT // 16):
            acc[r, pl.ds(c * 16, 16)] = jnp.zeros((16,), jnp.float32)
        return 0

    lax.fori_loop(0, NPW + 1, zero_acc, 0)

    def zero_eids(i, _):
        eids[pl.ds(i * 16, 16)] = jnp.zeros((16,), jnp.int32)
        return 0

    lax.fori_loop(0, (SCH + 16) // 16, zero_eids, 0)

    iota = lax.iota(jnp.int32, 16)

    def chunk(ch, _):
        off = ch * SCH
        pltpu.sync_copy(dst_hbm.at[pl.ds(off, SCH)], dch)

        def filt(i, cnt):
            d = dch[pl.ds(i * 16, 16)]
            msk = jnp.logical_and(d >= lo, d < lo + NPW)
            e = off + i * 16 + iota
            s = jnp.where(msk, 1, 0)
            for k in (1, 2, 4, 8):
                sh = jnp.take(s, jnp.maximum(iota - k, 0))
                s = s + jnp.where(iota >= k, sh, 0)
            cum = s
            pos = cnt + cum - 1
            plsc.store_scatter(eids, [pos], e, mask=msk)
            plsc.store_scatter(dvals, [pos], d, mask=msk)
            return cnt + cum[15]

        cnt = lax.fori_loop(0, SCH // 16, filt, 0)
        ngrp = (cnt + GRP - 1) // GRP

        def grp(k, _):
            pltpu.async_copy(m_hbm.at[eids.at[pl.ds(k * GRP, GRP)]], mbuf, sem).wait()
            for g in range(GRP // 16):
                gbase = k * GRP + g * 16
                dvec = dvals[pl.ds(gbase, 16)]
                valid = (gbase + iota) < cnt
                dloc = jnp.where(valid, dvec - lo, NPW)
                for j in range(16):
                    dj = dloc[j]
                    for c in range(LAT // 16):
                        sl = pl.ds(c * 16, 16)
                        acc[dj, sl] = jnp.maximum(acc[dj, sl], mbuf[g * 16 + j, sl])
            return 0

        lax.fori_loop(0, ngrp, grp, 0)
        return 0

    lax.fori_loop(0, SNCH, chunk, 0)
    pltpu.sync_copy(acc.at[pl.ds(0, NPW)], out_hbm.at[pl.ds(lo, NPW)])


def _tc_node_transform(x, w1, b1):
    k = w1.shape[0] // 2

    def body(x_ref, w_ref, b_ref, a_ref, bb_ref):
        wt = w_ref[:k, :]
        wb = w_ref[k:, :]
        xv = x_ref[...]
        a_ref[...] = jnp.dot(xv, wt - wb, preferred_element_type=jnp.float32) + b_ref[...]
        bb_ref[...] = jnp.dot(xv, wb, preferred_element_type=jnp.float32)

    n = x.shape[0]
    return pl.pallas_call(
        body,
        out_shape=[
            jax.ShapeDtypeStruct((n, HID), jnp.float32),
            jax.ShapeDtypeStruct((n, HID), jnp.float32),
        ],
    )(x, w1, b1.reshape(1, HID))


_BE = 3200


def _tc_edge_mlp(pre, w2, b2, w3, b3):
    def body(p_ref, w2_ref, b2_ref, w3_ref, b3_ref, o_ref):
        h = jnp.maximum(p_ref[...], 0.0)
        h = jnp.dot(h, w2_ref[...], preferred_element_type=jnp.float32) + b2_ref[...]
        h = jnp.maximum(h, 0.0)
        h = jnp.dot(h, w3_ref[...], preferred_element_type=jnp.float32) + b3_ref[...]
        o_ref[...] = jnp.maximum(h, 0.0)

    return pl.pallas_call(
        body,
        grid=(E // _BE,),
        in_specs=[
            pl.BlockSpec((_BE, HID), lambda i: (i, 0)),
            pl.BlockSpec((HID, HID), lambda i: (0, 0)),
            pl.BlockSpec((1, HID), lambda i: (0, 0)),
            pl.BlockSpec((HID, LAT), lambda i: (0, 0)),
            pl.BlockSpec((1, LAT), lambda i: (0, 0)),
        ],
        out_specs=pl.BlockSpec((_BE, LAT), lambda i: (i, 0)),
        out_shape=jax.ShapeDtypeStruct((E, LAT), jnp.float32),
    )(pre, w2, b2.reshape(1, HID), w3, b3.reshape(1, LAT))


def _tc_pool_head(h, u_pad, w1p, b1, w2, b2, w3p, b3p):
    def body(h_ref, u_ref, w1_ref, b1_ref, w2_ref, b2_ref, w3_ref, b3_ref, o_ref):
        hv = h_ref[...]
        s = jnp.sum(hv, axis=0, keepdims=True)
        mx = jnp.max(hv, axis=0, keepdims=True)
        mean = s * (1.0 / N)
        o = (jnp.dot(s, w1_ref[:LAT, :], preferred_element_type=jnp.float32)
             + jnp.dot(mean, w1_ref[LAT:2 * LAT, :], preferred_element_type=jnp.float32)
             + jnp.dot(mx, w1_ref[2 * LAT:3 * LAT, :], preferred_element_type=jnp.float32)
             + jnp.dot(u_ref[...], w1_ref[3 * LAT:, :], preferred_element_type=jnp.float32)
             + b1_ref[...])
        o = jnp.maximum(o, 0.0)
        o = jnp.maximum(jnp.dot(o, w2_ref[...], preferred_element_type=jnp.float32)
                        + b2_ref[...], 0.0)
        o_ref[...] = jnp.dot(o, w3_ref[...], preferred_element_type=jnp.float32) + b3_ref[...]

    return pl.pallas_call(
        body,
        out_shape=jax.ShapeDtypeStruct((1, 128), jnp.float32),
    )(h, u_pad, w1p, b1, w2, b2, w3p, b3p)


def kernel(x, pos, edge_index, batch, u,
           l0W1, l0b1, l0W2, l0b2, l0W3, l0b3,
           l1W1, l1b1, l1W2, l1b2, l1W3, l1b3,
           linW1, linb1, linW2, linb2, linW3, linb3):
    src = edge_index[0]
    dst = edge_index[1]

    def edge_layer(h, w1, b1, w2, b2, w3, b3):
        a, b = _tc_node_transform(h, w1, b1)
        pre = _sc_gather_add(a, b, dst, src)
        m = _tc_edge_mlp(pre, w2, b2, w3, b3)
        hn = _sc_scatter_max(dst, m)
        return hn[:N]

    h = edge_layer(x, l0W1, l0b1, l0W2, l0b2, l0W3, l0b3)
    h = edge_layer(h, l1W1, l1b1, l1W2, l1b2, l1W3, l1b3)

    u_pad = jnp.pad(u, ((0, 0), (0, LAT - u.shape[1])))
    w1p = jnp.pad(linW1, ((0, 4 * LAT - linW1.shape[0]), (0, 0)))
    w3p = jnp.pad(linW3, ((0, 0), (0, 128 - linW3.shape[1])))
    b3p = jnp.pad(linb3.reshape(1, -1), ((0, 0), (0, 128 - linb3.shape[0])))
    out = _tc_pool_head(h, u_pad, w1p, linb1.reshape(1, LAT),
                        linW2, linb2.reshape(1, LAT), w3p, b3p)
    return out[:, :2]


# speedup vs baseline: 2.9508x; 2.9508x over previous
import functools

import jax
import jax.numpy as jnp
from jax import lax
from jax.experimental import pallas as pl
from jax.experimental.pallas import tpu as pltpu
from jax.experimental.pallas import tpu_sc as plsc

N = 10000
E = 320000
D = 128
HID = 128
LAT = 64

NC = 2
NS = 16
NW = NC * NS

NPW = 320
NOUT = NW * NPW
SCH = 6400
SNCH = E // SCH
BCAP = SCH + 256
EPB = E + 1024
GRP = 128


EPW = E // NW
GCH = 128
GNCH = EPW // GCH
GTL = EPW - GNCH * GCH


@functools.partial(
    pl.kernel,
    out_type=[
        jax.ShapeDtypeStruct((E, D), jnp.float32),
        jax.ShapeDtypeStruct((E, D), jnp.float32),
    ],
    mesh=plsc.VectorSubcoreMesh(core_axis_name="c", subcore_axis_name="s", num_cores=NC, num_subcores=NS),
    compiler_params=pltpu.CompilerParams(needs_layout_passes=False),
    scratch_types=[
        pltpu.VMEM((GCH,), jnp.int32),
        pltpu.VMEM((GCH,), jnp.int32),
        pltpu.VMEM((GCH, D), jnp.float32),
        pltpu.VMEM((GCH, D), jnp.float32),
        pltpu.VMEM((GTL,), jnp.int32),
        pltpu.VMEM((GTL,), jnp.int32),
        pltpu.VMEM((GTL, D), jnp.float32),
        pltpu.VMEM((GTL, D), jnp.float32),
        pltpu.SemaphoreType.DMA,
        pltpu.SemaphoreType.DMA,
    ],
)
def _sc_gather2(t_hbm, dst_hbm, src_hbm, xi_hbm, xj_hbm,
                di_v, si_v, ibuf, jbuf, di_t, si_t, ibt, jbt, sem_i, sem_j):
    wid = lax.axis_index("s") * NC + lax.axis_index("c")
    base = wid * EPW

    def chunk(i, _):
        off = base + i * GCH
        pltpu.sync_copy(dst_hbm.at[pl.ds(off, GCH)], di_v)
        pltpu.sync_copy(src_hbm.at[pl.ds(off, GCH)], si_v)
        cp_i = pltpu.async_copy(t_hbm.at[di_v], ibuf, sem_i)
        cp_j = pltpu.async_copy(t_hbm.at[si_v], jbuf, sem_j)
        cp_i.wait()
        cp_j.wait()
        pltpu.sync_copy(ibuf, xi_hbm.at[pl.ds(off, GCH)])
        pltpu.sync_copy(jbuf, xj_hbm.at[pl.ds(off, GCH)])
        return 0

    lax.fori_loop(0, GNCH, chunk, 0)
    off = base + GNCH * GCH
    pltpu.sync_copy(dst_hbm.at[pl.ds(off, GTL)], di_t)
    pltpu.sync_copy(src_hbm.at[pl.ds(off, GTL)], si_t)
    cp_i = pltpu.async_copy(t_hbm.at[di_t], ibt, sem_i)
    cp_j = pltpu.async_copy(t_hbm.at[si_t], jbt, sem_j)
    cp_i.wait()
    cp_j.wait()
    pltpu.sync_copy(ibt, xi_hbm.at[pl.ds(off, GTL)])
    pltpu.sync_copy(jbt, xj_hbm.at[pl.ds(off, GTL)])


@functools.partial(
    pl.kernel,
    out_type=[
        jax.ShapeDtypeStruct((NW * EPB,), jnp.int32),
        jax.ShapeDtypeStruct((NW * EPB,), jnp.int32),
        jax.ShapeDtypeStruct((NW * 16,), jnp.int32),
    ],
    mesh=plsc.VectorSubcoreMesh(core_axis_name="c", subcore_axis_name="s", num_cores=NC, num_subcores=NS),
    compiler_params=pltpu.CompilerParams(needs_layout_passes=False),
    scratch_types=[
        pltpu.VMEM((SCH,), jnp.int32),
        pltpu.VMEM((SCH,), jnp.int32),
        pltpu.VMEM((BCAP,), jnp.int32),
        pltpu.VMEM((BCAP,), jnp.int32),
        pltpu.VMEM((16,), jnp.int32),
        pltpu.SemaphoreType.DMA,
        pltpu.SemaphoreType.DMA,
    ],
)
def _sc_bucketize(dst_hbm, ebuck, dbuck, counts, dchA, dchB, ebuf, dbuf, cst,
                  semA, semB):
    wid = lax.axis_index("s") * NC + lax.axis_index("c")
    lo = wid * NPW
    wbase = wid * EPB
    iota = lax.iota(jnp.int32, 16)

    def zero_buf(i, _):
        ebuf[pl.ds(i * 16, 16)] = jnp.zeros((16,), jnp.int32)
        dbuf[pl.ds(i * 16, 16)] = jnp.zeros((16,), jnp.int32)
        return 0

    lax.fori_loop(0, BCAP // 16, zero_buf, 0)

    def process(dch, ch, tot, res):
        off = ch * SCH

        def filt(i, pos):
            d = dch[pl.ds(i * 16, 16)]
            msk = jnp.logical_and(d >= lo, d < lo + NPW)
            e = off + i * 16 + iota
            s = jnp.where(msk, 1, 0)
            for k in (1, 2, 4, 8):
                sh = jnp.take(s, jnp.maximum(iota - k, 0))
                s = s + jnp.where(iota >= k, sh, 0)
            p = pos + s - 1
            plsc.store_scatter(ebuf, [p], e, mask=msk)
            plsc.store_scatter(dbuf, [p], d - lo, mask=msk)
            return pos + s[15]

        pos = lax.fori_loop(0, SCH // 16, filt, res)
        nfull = (pos // 8) * 8
        nblk = (pos + 511) // 512

        def flush(b, _):
            sl = pl.ds(b * 512, 512)
            ob = pl.multiple_of(wbase + tot + b * 512, 8)
            pltpu.sync_copy(ebuf.at[sl], ebuck.at[pl.ds(ob, 512)])
            pltpu.sync_copy(dbuf.at[sl], dbuck.at[pl.ds(ob, 512)])
            return 0

        lax.fori_loop(0, nblk, flush, 0)
        rem_e = ebuf[pl.ds(nfull, 16)]
        rem_d = dbuf[pl.ds(nfull, 16)]
        ebuf[pl.ds(0, 16)] = rem_e
        dbuf[pl.ds(0, 16)] = rem_d
        return tot + nfull, pos - nfull

    pltpu.async_copy(dst_hbm.at[pl.ds(0, SCH)], dchA, semA)

    def pair(p, carry):
        tot, res = carry
        off_b = jnp.minimum(2 * p + 1, SNCH - 1) * SCH
        cpB = pltpu.async_copy(dst_hbm.at[pl.ds(off_b, SCH)], dchB, semB)
        pltpu.make_async_copy(dst_hbm.at[pl.ds(0, SCH)], dchA, semA).wait()
        tot, res = process(dchA, 2 * p, tot, res)
        off_a = jnp.minimum(2 * p + 2, SNCH - 1) * SCH
        pltpu.async_copy(dst_hbm.at[pl.ds(off_a, SCH)], dchA, semA)
        cpB.wait()
        tot, res = process(dchB, 2 * p + 1, tot, res)
        return tot, res

    tot, res = lax.fori_loop(0, SNCH // 2, pair, (0, 0))
    pltpu.make_async_copy(dst_hbm.at[pl.ds(0, SCH)], dchA, semA).wait()
    for i in range(8):
        ebuf[pl.ds(res + i * 16, 16)] = jnp.zeros((16,), jnp.int32)
        dbuf[pl.ds(res + i * 16, 16)] = jnp.zeros((16,), jnp.int32)
    of = pl.multiple_of(wbase + tot, 8)
    pltpu.sync_copy(ebuf.at[pl.ds(0, 512)], ebuck.at[pl.ds(of, 512)])
    pltpu.sync_copy(dbuf.at[pl.ds(0, 512)], dbuck.at[pl.ds(of, 512)])
    cst[pl.ds(0, 16)] = jnp.broadcast_to(tot + res, (16,)).astype(jnp.int32)
    pltpu.sync_copy(cst, counts.at[pl.ds(wid * 16, 16)])


@functools.partial(
    pl.kernel,
    out_type=jax.ShapeDtypeStruct((NOUT, 2 * LAT), jnp.float32),
    mesh=plsc.VectorSubcoreMesh(core_axis_name="c", subcore_axis_name="s", num_cores=NC, num_subcores=NS),
    compiler_params=pltpu.CompilerParams(needs_layout_passes=False, use_tc_tiling_on_sc=False),
    scratch_types=[
        pltpu.VMEM((NPW + 1, 2 * LAT), jnp.float32),
        pltpu.VMEM((GRP,), jnp.int32),
        pltpu.VMEM((GRP,), jnp.int32),
        pltpu.VMEM((GRP, 2 * LAT), jnp.float32),
        pltpu.VMEM((GRP,), jnp.int32),
        pltpu.VMEM((GRP,), jnp.int32),
        pltpu.VMEM((GRP, 2 * LAT), jnp.float32),
        pltpu.VMEM((16,), jnp.int32),
        pltpu.SemaphoreType.DMA,
        pltpu.SemaphoreType.DMA,
        pltpu.SemaphoreType.DMA,
        pltpu.SemaphoreType.DMA,
    ],
)
def _sc_scatter_max(ebuck, dbuck, counts, m_hbm, out_hbm,
                    acc, ebA, dbA, mbA, ebB, dbB, mbB, cv,
                    semeA, semmA, semeB, semmB):
    wid = lax.axis_index("s") * NC + lax.axis_index("c")
    lo = wid * NPW
    wbase = wid * EPB
    iota = lax.iota(jnp.int32, 16)

    def zero_acc(r, _):
        for c in range(2 * LAT // 16):
            acc[r, pl.ds(c * 16, 16)] = jnp.zeros((16,), jnp.float32)
        return 0

    lax.fori_loop(0, NPW + 1, zero_acc, 0)

    pltpu.sync_copy(counts.at[pl.ds(wid * 16, 16)], cv)
    cnt = cv[pl.ds(0, 16)][0]
    ngrp = (cnt + GRP - 1) // GRP

    def fetch(k, eb, db, mb, seme, semm):
        ke = jnp.maximum(jnp.minimum(k, ngrp - 1), 0) * GRP
        pltpu.async_copy(ebuck.at[pl.ds(wbase + ke, GRP)], eb, seme).wait()
        pltpu.sync_copy(dbuck.at[pl.ds(wbase + ke, GRP)], db)
        return pltpu.async_copy(m_hbm.at[eb], mb, semm)

    def process(k, db, mb):
        for g in range(GRP // 16):
            gbase = k * GRP + g * 16
            dvec = db[pl.ds(g * 16, 16)]
            valid = (gbase + iota) < cnt
            dloc = jnp.where(valid, dvec, NPW)
            for j in range(16):
                dj = dloc[j]
                for c in range(LAT // 16):
                    sl = pl.ds(c * 16, 16)
                    acc[dj, sl] = jnp.maximum(acc[dj, sl], mb[g * 16 + j, sl])
        return 0

    fetch(0, ebA, dbA, mbA, semeA, semmA)

    def pair(p, _):
        k = 2 * p
        cpB = fetch(k + 1, ebB, dbB, mbB, semeB, semmB)
        pltpu.make_async_copy(m_hbm.at[ebA], mbA, semmA).wait()

        @pl.when(k < ngrp)
        def _():
            process(k, dbA, mbA)

        fetch(k + 2, ebA, dbA, mbA, semeA, semmA)
        cpB.wait()

        @pl.when(k + 1 < ngrp)
        def _():
            process(k + 1, dbB, mbB)

        return 0

    npair = (ngrp + 1) // 2
    lax.fori_loop(0, npair, pair, 0)
    pltpu.make_async_copy(m_hbm.at[ebA], mbA, semmA).wait()
    pltpu.sync_copy(acc.at[pl.ds(0, NPW)], out_hbm.at[pl.ds(lo, NPW)])


_BE = 3200


def _tc_edge_mlp(xi, xj, k, w1, b1, w2, b2, w3, b3):
    def body(xi_ref, xj_ref, w1_ref, b1_ref, w2_ref, b2_ref, w3_ref, b3_ref, o_ref):
        a = xi_ref[:, :k]
        b = xj_ref[:, :k]
        h = jnp.concatenate([a, b - a], axis=1)
        h = jnp.dot(h, w1_ref[...], preferred_element_type=jnp.float32) + b1_ref[...]
        h = jnp.maximum(h, 0.0)
        h = jnp.dot(h, w2_ref[...], preferred_element_type=jnp.float32) + b2_ref[...]
        h = jnp.maximum(h, 0.0)
        h = jnp.dot(h, w3_ref[...], preferred_element_type=jnp.float32) + b3_ref[...]
        o_ref[...] = jnp.maximum(h, 0.0)

    w3p = jnp.pad(w3, ((0, 0), (0, 2 * LAT - w3.shape[1])))
    b3p = jnp.pad(b3.reshape(1, LAT), ((0, 0), (0, LAT)))
    return pl.pallas_call(
        body,
        grid=(E // _BE,),
        in_specs=[
            pl.BlockSpec((_BE, D), lambda i: (i, 0)),
            pl.BlockSpec((_BE, D), lambda i: (i, 0)),
            pl.BlockSpec((2 * k, HID), lambda i: (0, 0)),
            pl.BlockSpec((1, HID), lambda i: (0, 0)),
            pl.BlockSpec((HID, HID), lambda i: (0, 0)),
            pl.BlockSpec((1, HID), lambda i: (0, 0)),
            pl.BlockSpec((HID, 2 * LAT), lambda i: (0, 0)),
            pl.BlockSpec((1, 2 * LAT), lambda i: (0, 0)),
        ],
        out_specs=pl.BlockSpec((_BE, 2 * LAT), lambda i: (i, 0)),
        out_shape=jax.ShapeDtypeStruct((E, 2 * LAT), jnp.float32),
    )(xi, xj, w1, b1.reshape(1, HID), w2, b2.reshape(1, HID), w3p, b3p)


def _tc_pool(h):
    def body(h_ref, s_ref, m_ref):
        hv = h_ref[...]
        s_ref[...] = jnp.sum(hv, axis=0, keepdims=True)
        m_ref[...] = jnp.max(hv, axis=0, keepdims=True)

    return pl.pallas_call(
        body,
        out_shape=[
            jax.ShapeDtypeStruct((1, LAT), jnp.float32),
            jax.ShapeDtypeStruct((1, LAT), jnp.float32),
        ],
    )(h)


def kernel(x, pos, edge_index, batch, u,
           l0W1, l0b1, l0W2, l0b2, l0W3, l0b3,
           l1W1, l1b1, l1W2, l1b2, l1W3, l1b3,
           linW1, linb1, linW2, linb2, linW3, linb3):
    src = edge_index[0]
    dst = edge_index[1]
    ebuck, dbuck, cnts = _sc_bucketize(dst)

    xpad = jnp.pad(x, ((0, NOUT - N), (0, 0)))

    def edge_layer(t, k, w1, b1, w2, b2, w3, b3):
        xi, xj = _sc_gather2(t, dst, src)
        m = _tc_edge_mlp(xi, xj, k, w1, b1, w2, b2, w3, b3)
        return _sc_scatter_max(ebuck, dbuck, cnts, m)

    h = edge_layer(xpad, D, l0W1, l0b1, l0W2, l0b2, l0W3, l0b3)
    h = edge_layer(h, LAT, l1W1, l1b1, l1W2, l1b2, l1W3, l1b3)

    s, mx = _tc_pool(h[:N, :LAT])
    meanp = s / jnp.float32(N)
    pooled = jnp.concatenate([s, meanp, mx, u], axis=1)
    o = jax.nn.relu(pooled @ linW1 + linb1)
    o = jax.nn.relu(o @ linW2 + linb2)
    return o @ linW3 + linb3


# speedup vs baseline: 3.1771x; 1.0767x over previous
import functools

import jax
import jax.numpy as jnp
from jax import lax
from jax.experimental import pallas as pl
from jax.experimental.pallas import tpu as pltpu
from jax.experimental.pallas import tpu_sc as plsc

N = 10000
E = 320000
D = 128
HID = 128
LAT = 64

NC = 2
NS = 16
NW = NC * NS

NPW = 320
NOUT = NW * NPW
SCH = 6400
SNCH = E // SCH
BCAP = SCH + 256
EPB = E + 1024
GRP = 128
SG = 8


EPW = E // NW
GCH = 128
GNCH = EPW // GCH
GTL = EPW - GNCH * GCH


@functools.partial(
    pl.kernel,
    out_type=[
        jax.ShapeDtypeStruct((E, D), jnp.float32),
        jax.ShapeDtypeStruct((E, D), jnp.float32),
    ],
    mesh=plsc.VectorSubcoreMesh(core_axis_name="c", subcore_axis_name="s", num_cores=NC, num_subcores=NS),
    compiler_params=pltpu.CompilerParams(needs_layout_passes=False),
    scratch_types=[
        pltpu.VMEM((GCH,), jnp.int32),
        pltpu.VMEM((GCH,), jnp.int32),
        pltpu.VMEM((GCH, D), jnp.float32),
        pltpu.VMEM((GCH, D), jnp.float32),
        pltpu.VMEM((GCH,), jnp.int32),
        pltpu.VMEM((GCH,), jnp.int32),
        pltpu.VMEM((GCH, D), jnp.float32),
        pltpu.VMEM((GCH, D), jnp.float32),
        pltpu.VMEM((GTL,), jnp.int32),
        pltpu.VMEM((GTL,), jnp.int32),
        pltpu.VMEM((GTL, D), jnp.float32),
        pltpu.VMEM((GTL, D), jnp.float32),
        pltpu.SemaphoreType.DMA,
        pltpu.SemaphoreType.DMA,
        pltpu.SemaphoreType.DMA,
        pltpu.SemaphoreType.DMA,
    ],
)
def _sc_gather2(t_hbm, dst_hbm, src_hbm, xi_hbm, xj_hbm,
                diA, siA, ibA, jbA, diB, siB, ibB, jbB,
                di_t, si_t, ibt, jbt, semA, semB, semWA, semWB):
    wid = lax.axis_index("s") * NC + lax.axis_index("c")
    base = wid * EPW

    def fire(ch, di, si, ib, jb, sem):
        off = base + ch * GCH
        pltpu.sync_copy(dst_hbm.at[pl.ds(off, GCH)], di)
        pltpu.sync_copy(src_hbm.at[pl.ds(off, GCH)], si)
        pltpu.async_copy(t_hbm.at[di], ib, sem)
        pltpu.async_copy(t_hbm.at[si], jb, sem)

    def wait_g(ib, jb, sem):
        pltpu.make_async_copy(t_hbm.at[pl.ds(0, GCH)], ib, sem).wait()
        pltpu.make_async_copy(t_hbm.at[pl.ds(0, GCH)], jb, sem).wait()

    def wb(ch, ib, jb):
        off = base + ch * GCH
        pltpu.sync_copy(ib, xi_hbm.at[pl.ds(off, GCH)])
        pltpu.sync_copy(jb, xj_hbm.at[pl.ds(off, GCH)])

    fire(0, diA, siA, ibA, jbA, semA)

    def pair(p, _):
        k = 2 * p
        fire(jnp.minimum(k + 1, GNCH - 1), diB, siB, ibB, jbB, semB)
        wait_g(ibA, jbA, semA)
        wb(k, ibA, jbA)
        fire(jnp.minimum(k + 2, GNCH - 1), diA, siA, ibA, jbA, semA)
        wait_g(ibB, jbB, semB)
        wb(k + 1, ibB, jbB)
        return 0

    lax.fori_loop(0, GNCH // 2, pair, 0)
    wait_g(ibA, jbA, semA)
    off = base + GNCH * GCH
    pltpu.sync_copy(dst_hbm.at[pl.ds(off, GTL)], di_t)
    pltpu.sync_copy(src_hbm.at[pl.ds(off, GTL)], si_t)
    cp_i = pltpu.async_copy(t_hbm.at[di_t], ibt, semWA)
    cp_j = pltpu.async_copy(t_hbm.at[si_t], jbt, semWB)
    cp_i.wait()
    cp_j.wait()
    pltpu.sync_copy(ibt, xi_hbm.at[pl.ds(off, GTL)])
    pltpu.sync_copy(jbt, xj_hbm.at[pl.ds(off, GTL)])


@functools.partial(
    pl.kernel,
    out_type=[
        jax.ShapeDtypeStruct((NW * EPB,), jnp.int32),
        jax.ShapeDtypeStruct((NW * EPB,), jnp.int32),
        jax.ShapeDtypeStruct((NW * 16,), jnp.int32),
    ],
    mesh=plsc.VectorSubcoreMesh(core_axis_name="c", subcore_axis_name="s", num_cores=NC, num_subcores=NS),
    compiler_params=pltpu.CompilerParams(needs_layout_passes=False),
    scratch_types=[
        pltpu.VMEM((SCH,), jnp.int32),
        pltpu.VMEM((SCH,), jnp.int32),
        pltpu.VMEM((BCAP,), jnp.int32),
        pltpu.VMEM((BCAP,), jnp.int32),
        pltpu.VMEM((16,), jnp.int32),
        pltpu.SemaphoreType.DMA,
        pltpu.SemaphoreType.DMA,
    ],
)
def _sc_bucketize(dst_hbm, ebuck, dbuck, counts, dchA, dchB, ebuf, dbuf, cst,
                  semA, semB):
    wid = lax.axis_index("s") * NC + lax.axis_index("c")
    lo = wid * NPW
    wbase = wid * EPB
    iota = lax.iota(jnp.int32, 16)

    def zero_buf(i, _):
        ebuf[pl.ds(i * 16, 16)] = jnp.zeros((16,), jnp.int32)
        dbuf[pl.ds(i * 16, 16)] = jnp.zeros((16,), jnp.int32)
        return 0

    lax.fori_loop(0, BCAP // 16, zero_buf, 0)

    def process(dch, ch, tot, res):
        off = ch * SCH

        def filt(i, pos):
            d = dch[pl.ds(i * 16, 16)]
            msk = jnp.logical_and(d >= lo, d < lo + NPW)
            e = off + i * 16 + iota
            s = jnp.where(msk, 1, 0)
            for k in (1, 2, 4, 8):
                sh = jnp.take(s, jnp.maximum(iota - k, 0))
                s = s + jnp.where(iota >= k, sh, 0)
            p = pos + s - 1
            plsc.store_scatter(ebuf, [p], e, mask=msk)
            plsc.store_scatter(dbuf, [p], d - lo, mask=msk)
            return pos + s[15]

        pos = lax.fori_loop(0, SCH // 16, filt, res)
        nfull = (pos // 8) * 8
        nblk = (pos + 511) // 512

        def flush(b, _):
            sl = pl.ds(b * 512, 512)
            ob = pl.multiple_of(wbase + tot + b * 512, 8)
            pltpu.sync_copy(ebuf.at[sl], ebuck.at[pl.ds(ob, 512)])
            pltpu.sync_copy(dbuf.at[sl], dbuck.at[pl.ds(ob, 512)])
            return 0

        lax.fori_loop(0, nblk, flush, 0)
        rem_e = ebuf[pl.ds(nfull, 16)]
        rem_d = dbuf[pl.ds(nfull, 16)]
        ebuf[pl.ds(0, 16)] = rem_e
        dbuf[pl.ds(0, 16)] = rem_d
        return tot + nfull, pos - nfull

    pltpu.async_copy(dst_hbm.at[pl.ds(0, SCH)], dchA, semA)

    def pair(p, carry):
        tot, res = carry
        off_b = jnp.minimum(2 * p + 1, SNCH - 1) * SCH
        cpB = pltpu.async_copy(dst_hbm.at[pl.ds(off_b, SCH)], dchB, semB)
        pltpu.make_async_copy(dst_hbm.at[pl.ds(0, SCH)], dchA, semA).wait()
        tot, res = process(dchA, 2 * p, tot, res)
        off_a = jnp.minimum(2 * p + 2, SNCH - 1) * SCH
        pltpu.async_copy(dst_hbm.at[pl.ds(off_a, SCH)], dchA, semA)
        cpB.wait()
        tot, res = process(dchB, 2 * p + 1, tot, res)
        return tot, res

    tot, res = lax.fori_loop(0, SNCH // 2, pair, (0, 0))
    pltpu.make_async_copy(dst_hbm.at[pl.ds(0, SCH)], dchA, semA).wait()
    for i in range(8):
        ebuf[pl.ds(res + i * 16, 16)] = jnp.zeros((16,), jnp.int32)
        dbuf[pl.ds(res + i * 16, 16)] = jnp.zeros((16,), jnp.int32)
    of = pl.multiple_of(wbase + tot, 8)
    pltpu.sync_copy(ebuf.at[pl.ds(0, 512)], ebuck.at[pl.ds(of, 512)])
    pltpu.sync_copy(dbuf.at[pl.ds(0, 512)], dbuck.at[pl.ds(of, 512)])
    cst[pl.ds(0, 16)] = jnp.broadcast_to(tot + res, (16,)).astype(jnp.int32)
    pltpu.sync_copy(cst, counts.at[pl.ds(wid * 16, 16)])


@functools.partial(
    pl.kernel,
    out_type=jax.ShapeDtypeStruct((NOUT, 2 * LAT), jnp.float32),
    mesh=plsc.VectorSubcoreMesh(core_axis_name="c", subcore_axis_name="s", num_cores=NC, num_subcores=NS),
    compiler_params=pltpu.CompilerParams(needs_layout_passes=False, use_tc_tiling_on_sc=False),
    scratch_types=[
        pltpu.VMEM((NPW + 1, 2 * LAT), jnp.float32),
        pltpu.VMEM((GRP,), jnp.int32),
        pltpu.VMEM((GRP,), jnp.int32),
        pltpu.VMEM((GRP, 2 * LAT), jnp.float32),
        pltpu.VMEM((GRP,), jnp.int32),
        pltpu.VMEM((GRP,), jnp.int32),
        pltpu.VMEM((GRP, 2 * LAT), jnp.float32),
        pltpu.VMEM((16,), jnp.int32),
        pltpu.SemaphoreType.DMA,
        pltpu.SemaphoreType.DMA,
        pltpu.SemaphoreType.DMA,
        pltpu.SemaphoreType.DMA,
    ],
)
def _sc_scatter_max(ebuck, dbuck, counts, m_hbm, out_hbm,
                    acc, ebA, dbA, mbA, ebB, dbB, mbB, cv,
                    semeA, semmA, semeB, semmB):
    wid = lax.axis_index("s") * NC + lax.axis_index("c")
    lo = wid * NPW
    wbase = wid * EPB
    iota = lax.iota(jnp.int32, 16)

    def zero_acc(r, _):
        for c in range(2 * LAT // 16):
            acc[r, pl.ds(c * 16, 16)] = jnp.zeros((16,), jnp.float32)
        return 0

    lax.fori_loop(0, NPW + 1, zero_acc, 0)

    pltpu.sync_copy(counts.at[pl.ds(wid * 16, 16)], cv)
    cnt = cv[pl.ds(0, 16)][0]
    ngrp = (cnt + GRP - 1) // GRP

    def fetch(k, eb, db, mb, seme, semm):
        ke = jnp.maximum(jnp.minimum(k, ngrp - 1), 0) * GRP
        pltpu.async_copy(ebuck.at[pl.ds(wbase + ke, GRP)], eb, seme).wait()
        pltpu.sync_copy(dbuck.at[pl.ds(wbase + ke, GRP)], db)
        return pltpu.async_copy(m_hbm.at[eb], mb, semm)

    def process(k, db, mb):
        for g in range(GRP // 16):
            gbase = k * GRP + g * 16
            dvec = db[pl.ds(g * 16, 16)]
            valid = (gbase + iota) < cnt
            dloc = jnp.where(valid, dvec, NPW)
            for j in range(16):
                dj = dloc[j]
                for c in range(LAT // 16):
                    sl = pl.ds(c * 16, 16)
                    acc[dj, sl] = jnp.maximum(acc[dj, sl], mb[g * 16 + j, sl])
        return 0

    fetch(0, ebA, dbA, mbA, semeA, semmA)

    def pair(p, _):
        k = 2 * p
        cpB = fetch(k + 1, ebB, dbB, mbB, semeB, semmB)
        pltpu.make_async_copy(m_hbm.at[ebA], mbA, semmA).wait()

        @pl.when(k < ngrp)
        def _():
            process(k, dbA, mbA)

        fetch(k + 2, ebA, dbA, mbA, semeA, semmA)
        cpB.wait()

        @pl.when(k + 1 < ngrp)
        def _():
            process(k + 1, dbB, mbB)

        return 0

    npair = (ngrp + 1) // 2
    lax.fori_loop(0, npair, pair, 0)
    pltpu.make_async_copy(m_hbm.at[ebA], mbA, semmA).wait()
    pltpu.sync_copy(acc.at[pl.ds(0, NPW)], out_hbm.at[pl.ds(lo, NPW)])


_BE = 3200


def _tc_edge_mlp(xi, xj, k, w1, b1, w2, b2, w3, b3):
    def body(xi_ref, xj_ref, w1_ref, b1_ref, w2_ref, b2_ref, w3_ref, b3_ref, o_ref):
        a = xi_ref[:, :k]
        b = xj_ref[:, :k]
        h = jnp.concatenate([a, b - a], axis=1)
        h = jnp.dot(h, w1_ref[...], preferred_element_type=jnp.float32) + b1_ref[...]
        h = jnp.maximum(h, 0.0)
        h = jnp.dot(h, w2_ref[...], preferred_element_type=jnp.float32) + b2_ref[...]
        h = jnp.maximum(h, 0.0)
        h = jnp.dot(h, w3_ref[...], preferred_element_type=jnp.float32) + b3_ref[...]
        o_ref[...] = jnp.maximum(h, 0.0)

    w3p = jnp.pad(w3, ((0, 0), (0, 2 * LAT - w3.shape[1])))
    b3p = jnp.pad(b3.reshape(1, LAT), ((0, 0), (0, LAT)))
    return pl.pallas_call(
        body,
        grid=(E // _BE,),
        in_specs=[
            pl.BlockSpec((_BE, D), lambda i: (i, 0)),
            pl.BlockSpec((_BE, D), lambda i: (i, 0)),
            pl.BlockSpec((2 * k, HID), lambda i: (0, 0)),
            pl.BlockSpec((1, HID), lambda i: (0, 0)),
            pl.BlockSpec((HID, HID), lambda i: (0, 0)),
            pl.BlockSpec((1, HID), lambda i: (0, 0)),
            pl.BlockSpec((HID, 2 * LAT), lambda i: (0, 0)),
            pl.BlockSpec((1, 2 * LAT), lambda i: (0, 0)),
        ],
        out_specs=pl.BlockSpec((_BE, 2 * LAT), lambda i: (i, 0)),
        out_shape=jax.ShapeDtypeStruct((E, 2 * LAT), jnp.float32),
    )(xi, xj, w1, b1.reshape(1, HID), w2, b2.reshape(1, HID), w3p, b3p)


def _tc_pool(h):
    def body(h_ref, s_ref, m_ref):
        hv = h_ref[...]
        s_ref[...] = jnp.sum(hv, axis=0, keepdims=True)
        m_ref[...] = jnp.max(hv, axis=0, keepdims=True)

    return pl.pallas_call(
        body,
        out_shape=[
            jax.ShapeDtypeStruct((1, LAT), jnp.float32),
            jax.ShapeDtypeStruct((1, LAT), jnp.float32),
        ],
    )(h)


def kernel(x, pos, edge_index, batch, u,
           l0W1, l0b1, l0W2, l0b2, l0W3, l0b3,
           l1W1, l1b1, l1W2, l1b2, l1W3, l1b3,
           linW1, linb1, linW2, linb2, linW3, linb3):
    src = edge_index[0]
    dst = edge_index[1]
    ebuck, dbuck, cnts = _sc_bucketize(dst)

    xpad = jnp.pad(x, ((0, NOUT - N), (0, 0)))

    def edge_layer(t, k, w1, b1, w2, b2, w3, b3):
        xi, xj = _sc_gather2(t, dst, src)
        m = _tc_edge_mlp(xi, xj, k, w1, b1, w2, b2, w3, b3)
        return _sc_scatter_max(ebuck, dbuck, cnts, m)

    h = edge_layer(xpad, D, l0W1, l0b1, l0W2, l0b2, l0W3, l0b3)
    h = edge_layer(h, LAT, l1W1, l1b1, l1W2, l1b2, l1W3, l1b3)

    s, mx = _tc_pool(h[:N, :LAT])
    meanp = s / jnp.float32(N)
    pooled = jnp.concatenate([s, meanp, mx, u], axis=1)
    o = jax.nn.relu(pooled @ linW1 + linb1)
    o = jax.nn.relu(o @ linW2 + linb2)
    return o @ linW3 + linb3
